# Initial kernel scaffold; baseline (speedup 1.0000x reference)
import jax
import jax.numpy as jnp
from jax.experimental import pallas as pl


def kernel(nodes, adj, features, weight):
    raise NotImplementedError("write your pallas kernel here")


# speedup vs baseline: 2.7192x; 2.7192x over previous
import functools

import jax
import jax.numpy as jnp
from jax import lax
from jax.experimental import pallas as pl
from jax.experimental.pallas import tpu as pltpu
from jax.experimental.pallas import tpu_sc as plsc

N = 10000
DEG = 16
B = 1024
D = 128
EMB = 64
U = B * DEG

NC = 2
NS = 16
NW = NC * NS
L = 16

ROWS_PER_TILE = B // NW
U_PER_TILE = U // NW
RING = 8

NPAD = 10240


def _f32(x):
    return jnp.float32(x)


def _sc1_body(nodes_hbm, adj_hbm, feat_hbm,
              mask_hbm, sum1_hbm, adj2_hbm, colcount_hbm, uniq_hbm,
              nodes_v, neigh_v, zrow_v, idx_v, val_v, sum1_v, ring_v,
              nodes2_v, neigh2_v, nf_v, myuniq_v, adj2_v, af_v, ones_v,
              prescopy_v, uniq_v, tmp_v, tmpsum_v,
              present_sp, colcount_sp, uniq_sp,
              ring_sems, zsem, gsem):
    c = lax.axis_index("c")
    s = lax.axis_index("s")
    wid = c * NS + s
    base_b = wid * ROWS_PER_TILE

    def _zero16(i, _):
        zrow_v[pl.ds(i * L, L)] = jnp.zeros((L,), jnp.float32)
        return 0
    lax.fori_loop(0, NPAD // L, _zero16, 0)

    def _ones16(i, _):
        ones_v[0, pl.ds(i * L, L)] = jnp.full((L,), 1.0, jnp.float32)
        return 0
    lax.fori_loop(0, 4096 // L, _ones16, 0)

    zero_descs = []
    for r in range(ROWS_PER_TILE):
        d = pltpu.async_copy(
            zrow_v.at[pl.ds(0, N)],
            mask_hbm.at[pl.ds((base_b + r) * N, N)],
            zsem)
        zero_descs.append(d)

    pltpu.sync_copy(nodes_hbm.at[pl.ds(base_b, ROWS_PER_TILE)], nodes_v)
    pltpu.async_copy(adj_hbm.at[nodes_v], neigh_v, gsem).wait()

    lane = lax.iota(jnp.int32, L)
    one_idx = jnp.full((L,), 1, jnp.int32)

    def _lane_sum(x):
        tmpsum_v[pl.ds(0, L)] = jnp.zeros((L,), jnp.int32)
        plsc.addupdate_scatter(tmpsum_v, [one_idx], x)
        return plsc.load_gather(tmpsum_v, [one_idx])

    def _maskrow(i, _):
        nv = neigh_v[i]
        tmp_v[pl.ds(0, L)] = nv
        tmp_v[pl.ds(L, L)] = nv
        dupm = nv != nv
        for e in range(L - 1):
            ve = plsc.load_gather(tmp_v, [jnp.full((L,), L + e, jnp.int32)])
            dupm = dupm | ((nv == ve) & (lane > e))
        cnt = L - _lane_sum(dupm.astype(jnp.int32))
        inv = _f32(1.0) / cnt.astype(jnp.float32)
        flat = nv + (base_b + i) * N
        idx_v[0, pl.ds(i * L, L)] = flat
        val_v[0, pl.ds(i * L, L)] = inv
        return 0
    lax.fori_loop(0, ROWS_PER_TILE, _maskrow, 0)

    for r in range(RING):
        pltpu.async_copy(feat_hbm.at[neigh_v.at[r]], ring_v.at[r],
                         ring_sems.at[r])

    def _hop1_block(blk, _):
        for rr in range(RING):
            i = blk * RING + rr
            pltpu.make_async_copy(feat_hbm.at[neigh_v.at[rr]],
                                  ring_v.at[rr], ring_sems.at[rr]).wait()

            def _acc(j, acc):
                return tuple(acc[cc] + ring_v[rr, j, pl.ds(cc * L, L)]
                             for cc in range(D // L))
            acc0 = tuple(jnp.zeros((L,), jnp.float32) for _ in range(D // L))
            acc = lax.fori_loop(0, DEG, _acc, acc0)
            for cc in range(D // L):
                sum1_v[i, pl.ds(cc * L, L)] = acc[cc]

            @pl.when(i + RING < ROWS_PER_TILE)
            def _():
                pltpu.async_copy(feat_hbm.at[neigh_v.at[i + RING]],
                                 ring_v.at[rr], ring_sems.at[rr])
        return 0
    lax.fori_loop(0, ROWS_PER_TILE // RING, _hop1_block, 0)
    pltpu.sync_copy(sum1_v, sum1_hbm.at[pl.ds(base_b, ROWS_PER_TILE)])

    @pl.when(c == 0)
    def _core0():
        @pl.when(s == 0)
        def _():
            pltpu.sync_copy(zrow_v, present_sp)
        @pl.when(s == 1)
        def _():
            pltpu.sync_copy(zrow_v, colcount_sp)
        plsc.subcore_barrier()

        nb = B // NS
        pltpu.sync_copy(nodes_hbm.at[pl.ds(s * nb, nb)], nodes2_v)
        pltpu.async_copy(adj_hbm.at[nodes2_v], neigh2_v, gsem).wait()

        def _flat(i, _):
            nf_v[0, pl.ds(i * L, L)] = neigh2_v[i]
            return 0
        lax.fori_loop(0, nb, _flat, 0)
        pltpu.sync_copy(ones_v.at[0, pl.ds(0, nb * DEG)],
                        present_sp.at[nf_v.at[0]], add=True)
        plsc.subcore_barrier()

        @pl.when(s == 0)
        def _enumerate():
            pltpu.sync_copy(present_sp, prescopy_v)

            def _zu(i, _):
                uniq_v[pl.ds(i * L, L)] = jnp.zeros((L,), jnp.int32)
                return 0
            lax.fori_loop(0, U // L, _zu, 0)

            def _enum(i, base):
                pv = prescopy_v[pl.ds(i * L, L)]
                m = pv > _f32(0.0)
                mi = m.astype(jnp.int32)
                tmp_v[pl.ds(0, L)] = mi
                tmp_v[pl.ds(L, L)] = mi
                pre = jnp.zeros((L,), jnp.int32)
                for e in range(L - 1):
                    me = plsc.load_gather(
                        tmp_v, [jnp.full((L,), L + e, jnp.int32)])
                    pre = pre + jnp.where(lane > e, me, 0)
                rank = base + pre
                vals = lane + i * L
                plsc.store_scatter(uniq_v, [rank], vals, mask=m)
                tmp_v[pl.ds(0, L)] = pre + mi
                tot = plsc.load_gather(
                    tmp_v, [jnp.full((L,), L - 1, jnp.int32)])
                return base + tot
            lax.fori_loop(0, NPAD // L, _enum, jnp.zeros((L,), jnp.int32))
            pltpu.sync_copy(uniq_v, uniq_sp)
            pltpu.sync_copy(uniq_v, uniq_hbm)
        plsc.subcore_barrier()

        pltpu.sync_copy(uniq_sp.at[pl.ds(s * (U // NS), U // NS)], myuniq_v)
        pltpu.async_copy(adj_hbm.at[myuniq_v], adj2_v, gsem).wait()
        pltpu.sync_copy(adj2_v, adj2_hbm.at[pl.ds(s * (U // NS), U // NS)])

        for chunk in range(4):
            def _flat2(i, _):
                af_v[chunk, pl.ds(i * L, L)] = adj2_v[chunk * 256 + i]
                return 0
            lax.fori_loop(0, 256, _flat2, 0)
        for chunk in range(4):
            pltpu.sync_copy(ones_v.at[0],
                            colcount_sp.at[af_v.at[chunk]], add=True)
        plsc.subcore_barrier()

        @pl.when(s == 0)
        def _():
            pltpu.sync_copy(colcount_sp.at[pl.ds(0, N)], colcount_hbm)

    for d in zero_descs:
        d.wait()
    pltpu.sync_copy(val_v.at[0], mask_hbm.at[idx_v.at[0]])


_SC_PARAMS = pltpu.CompilerParams(needs_layout_passes=False,
                                  use_tc_tiling_on_sc=False)


def _sc1(nodes, adj, features):
    mesh = plsc.VectorSubcoreMesh(core_axis_name="c", subcore_axis_name="s")
    kfn = pl.kernel(
        _sc1_body,
        compiler_params=_SC_PARAMS,
        out_type=[
            jax.ShapeDtypeStruct((B * N,), jnp.float32),
            jax.ShapeDtypeStruct((B, D), jnp.float32),
            jax.ShapeDtypeStruct((U, DEG), jnp.int32),
            jax.ShapeDtypeStruct((N,), jnp.float32),
            jax.ShapeDtypeStruct((U,), jnp.int32),
        ],
        mesh=mesh,
        scratch_types=[
            pltpu.VMEM((ROWS_PER_TILE,), jnp.int32),
            pltpu.VMEM((ROWS_PER_TILE, DEG), jnp.int32),
            pltpu.VMEM((NPAD,), jnp.float32),
            pltpu.VMEM((1, ROWS_PER_TILE * DEG), jnp.int32),
            pltpu.VMEM((1, ROWS_PER_TILE * DEG), jnp.float32),
            pltpu.VMEM((ROWS_PER_TILE, D), jnp.float32),
            pltpu.VMEM((RING, DEG, D), jnp.float32),
            pltpu.VMEM((B // NS,), jnp.int32),
            pltpu.VMEM((B // NS, DEG), jnp.int32),
            pltpu.VMEM((1, B // NS * DEG), jnp.int32),
            pltpu.VMEM((U // NS,), jnp.int32),
            pltpu.VMEM((U // NS, DEG), jnp.int32),
            pltpu.VMEM((4, 4096), jnp.int32),
            pltpu.VMEM((1, 4096), jnp.float32),
            pltpu.VMEM((NPAD,), jnp.float32),
            pltpu.VMEM((U,), jnp.int32),
            pltpu.VMEM((2 * L,), jnp.int32),
            pltpu.VMEM((L,), jnp.int32),
            pltpu.VMEM_SHARED((NPAD,), jnp.float32),
            pltpu.VMEM_SHARED((NPAD,), jnp.float32),
            pltpu.VMEM_SHARED((U,), jnp.int32),
            pltpu.SemaphoreType.DMA((RING,)),
            pltpu.SemaphoreType.DMA,
            pltpu.SemaphoreType.DMA,
        ],
    )
    return kfn(nodes, adj, features)


def _sc2_body(scaled_hbm, adj2_hbm, agg2_hbm,
              adj2s_v, ring_v, out_v, ring_sems):
    c = lax.axis_index("c")
    s = lax.axis_index("s")
    wid = c * NS + s
    base_u = wid * U_PER_TILE

    pltpu.sync_copy(adj2_hbm.at[pl.ds(base_u, U_PER_TILE)], adj2s_v)

    for r in range(RING):
        pltpu.async_copy(scaled_hbm.at[adj2s_v.at[r]], ring_v.at[r],
                         ring_sems.at[r])

    def _block(blk, _):
        for rr in range(RING):
            u = blk * RING + rr
            pltpu.make_async_copy(scaled_hbm.at[adj2s_v.at[rr]],
                                  ring_v.at[rr], ring_sems.at[rr]).wait()

            def _acc(j, acc):
                return tuple(acc[cc] + ring_v[rr, j, pl.ds(cc * L, L)]
                             for cc in range(D // L))
            acc0 = tuple(jnp.zeros((L,), jnp.float32) for _ in range(D // L))
            acc = lax.fori_loop(0, DEG, _acc, acc0)
            for cc in range(D // L):
                out_v[u, pl.ds(cc * L, L)] = acc[cc]

            @pl.when(u + RING < U_PER_TILE)
            def _():
                pltpu.async_copy(scaled_hbm.at[adj2s_v.at[u + RING]],
                                 ring_v.at[rr], ring_sems.at[rr])
        return 0
    lax.fori_loop(0, U_PER_TILE // RING, _block, 0)

    pltpu.sync_copy(out_v, agg2_hbm.at[pl.ds(base_u, U_PER_TILE)])


def _sc2(scaled, adj2):
    mesh = plsc.VectorSubcoreMesh(core_axis_name="c", subcore_axis_name="s")
    kfn = pl.kernel(
        _sc2_body,
        compiler_params=_SC_PARAMS,
        out_type=jax.ShapeDtypeStruct((U, D), jnp.float32),
        mesh=mesh,
        scratch_types=[
            pltpu.VMEM((U_PER_TILE, DEG), jnp.int32),
            pltpu.VMEM((RING, DEG, D), jnp.float32),
            pltpu.VMEM((U_PER_TILE, D), jnp.float32),
            pltpu.SemaphoreType.DMA((RING,)),
        ],
    )
    return kfn(scaled, adj2)


def _scale_body(feat_ref, cc_ref, out_ref):
    cc = jnp.maximum(cc_ref[...], 1.0)
    out_ref[...] = feat_ref[...] * lax.rsqrt(cc)


def _tc_scale(features, colcount2d):
    blk = 400
    return pl.pallas_call(
        _scale_body,
        grid=(N // blk,),
        in_specs=[
            pl.BlockSpec((blk, D), lambda i: (i, 0)),
            pl.BlockSpec((blk, 1), lambda i: (i, 0)),
        ],
        out_specs=pl.BlockSpec((blk, D), lambda i: (i, 0)),
        out_shape=jax.ShapeDtypeStruct((N, D), jnp.float32),
    )(features, colcount2d)


def _mm1_body(sum1_ref, w_ref, out_ref):
    a = sum1_ref[...] * (1.0 / DEG)
    out_ref[...] = jnp.maximum(
        jnp.dot(a, w_ref[...], preferred_element_type=jnp.float32), 0.0)


def _tc_mm1(sum1, weight):
    return pl.pallas_call(
        _mm1_body,
        out_shape=jax.ShapeDtypeStruct((B, EMB), jnp.float32),
    )(sum1, weight)


def _mm2_body(agg_ref, w_ref, out_ref):
    a = agg_ref[...] * 0.25
    out_ref[...] = jnp.maximum(
        jnp.dot(a, w_ref[...], preferred_element_type=jnp.float32), 0.0)


def _tc_mm2(agg2, weight):
    blk = 1024
    return pl.pallas_call(
        _mm2_body,
        grid=(U // blk,),
        in_specs=[
            pl.BlockSpec((blk, D), lambda i: (i, 0)),
            pl.BlockSpec((D, EMB), lambda i: (0, 0)),
        ],
        out_specs=pl.BlockSpec((blk, EMB), lambda i: (i, 0)),
        out_shape=jax.ShapeDtypeStruct((U, EMB), jnp.float32),
    )(agg2, weight)


def kernel(nodes, adj, features, weight):
    mask_flat, sum1, adj2, colcount, _ = _sc1(nodes, adj, features)
    scaled = _tc_scale(features, colcount.reshape(N, 1))
    to_feats = _tc_mm1(sum1, weight)
    agg2 = _sc2(scaled, adj2)
    to_feats_neigh = _tc_mm2(agg2, weight)
    return to_feats, to_feats_neigh, mask_flat.reshape(B, N)


# speedup vs baseline: 3.6497x; 1.3422x over previous
import jax
import jax.numpy as jnp
from jax import lax
from jax.experimental import pallas as pl
from jax.experimental.pallas import tpu as pltpu
from jax.experimental.pallas import tpu_sc as plsc

N = 10000
DEG = 16
B = 1024
D = 128
EMB = 64
U = B * DEG

NC = 2
NS = 16
NW = NC * NS
L = 16

ROWS_PER_TILE = B // NW
U_PER_TILE = U // NW
NPAD = 10240
N_PER_TILE = NPAD // NW
ZELEMS = ROWS_PER_TILE * N

_SC_PARAMS = pltpu.CompilerParams(needs_layout_passes=False,
                                  use_tc_tiling_on_sc=False)


def _f32(x):
    return jnp.float32(x)


def _sc1_body(nodes_hbm, adj_hbm, featf_hbm,
              mask_hbm, sum1_hbm, colcount_hbm, uniq_hbm,
              nodes_v, neigh_v, zrow_v, idx_v, val_v, sum1_v, big_v,
              nodes2_v, neigh2_v, nf_v, myuniq_v, adj2_v, af_v, ones_v,
              prescopy_v, uniq_v, tmp_v, tmpsum_v,
              zregion_sp, present_sp, colcount_sp, uniq_sp,
              zsem, gsem):
    c = lax.axis_index("c")
    s = lax.axis_index("s")
    wid = c * NS + s
    base_b = wid * ROWS_PER_TILE

    def _zero16(i, _):
        zrow_v[pl.ds(i * L, L)] = jnp.zeros((L,), jnp.float32)
        return 0
    lax.fori_loop(0, NPAD // L, _zero16, 0)

    def _ones16(i, _):
        ones_v[0, pl.ds(i * L, L)] = jnp.full((L,), 1.0, jnp.float32)
        return 0
    lax.fori_loop(0, 4096 // L, _ones16, 0)

    zslice = ZELEMS // NS
    pltpu.sync_copy(zrow_v.at[pl.ds(0, NPAD)],
                    zregion_sp.at[pl.ds(s * zslice, NPAD)])
    pltpu.sync_copy(zrow_v.at[pl.ds(0, zslice - NPAD)],
                    zregion_sp.at[pl.ds(s * zslice + NPAD, zslice - NPAD)])
    plsc.subcore_barrier()

    zdesc = pltpu.async_copy(
        zregion_sp, mask_hbm.at[pl.ds(base_b * N, ROWS_PER_TILE * N)], zsem)

    pltpu.sync_copy(nodes_hbm.at[pl.ds(base_b, ROWS_PER_TILE)], nodes_v)
    pltpu.async_copy(adj_hbm.at[nodes_v], neigh_v, gsem).wait()

    lane = lax.iota(jnp.int32, L)
    one_idx = jnp.full((L,), 1, jnp.int32)

    def _lane_sum(x):
        tmpsum_v[pl.ds(0, L)] = jnp.zeros((L,), jnp.int32)
        plsc.addupdate_scatter(tmpsum_v, [one_idx], x)
        return plsc.load_gather(tmpsum_v, [one_idx])

    def _maskrow(i, _):
        nv = neigh_v[i]
        tmp_v[pl.ds(0, L)] = nv
        tmp_v[pl.ds(L, L)] = nv
        dupm = nv != nv
        for e in range(L - 1):
            ve = plsc.load_gather(tmp_v, [jnp.full((L,), L + e, jnp.int32)])
            dupm = dupm | ((nv == ve) & (lane > e))
        cnt = L - _lane_sum(dupm.astype(jnp.int32))
        inv = _f32(1.0) / cnt.astype(jnp.float32)
        flat = nv + (base_b + i) * N
        idx_v[0, pl.ds(i * L, L)] = flat
        val_v[0, pl.ds(i * L, L)] = inv
        nf_v[0, pl.ds(i * L, L)] = nv
        return 0
    lax.fori_loop(0, ROWS_PER_TILE, _maskrow, 0)

    half = ROWS_PER_TILE * DEG // 2
    for h in range(2):
        pltpu.async_copy(featf_hbm.at[nf_v.at[0, pl.ds(h * half, half)]],
                         big_v, gsem).wait()

        def _hop1(i, _):
            def _acc(j, acc):
                return tuple(acc[cc] + big_v[i * DEG + j, pl.ds(cc * L, L)]
                             for cc in range(D // L))
            acc0 = tuple(jnp.zeros((L,), jnp.float32) for _ in range(D // L))
            acc = lax.fori_loop(0, DEG, _acc, acc0)
            for cc in range(D // L):
                sum1_v[h * (ROWS_PER_TILE // 2) + i, pl.ds(cc * L, L)] = acc[cc]
            return 0
        lax.fori_loop(0, ROWS_PER_TILE // 2, _hop1, 0)
    pltpu.sync_copy(sum1_v, sum1_hbm.at[pl.ds(base_b, ROWS_PER_TILE)])

    @pl.when(c == 0)
    def _core0():
        @pl.when(s == 0)
        def _():
            pltpu.sync_copy(zrow_v, present_sp)
        @pl.when(s == 1)
        def _():
            pltpu.sync_copy(zrow_v, colcount_sp)
        plsc.subcore_barrier()

        nb = B // NS
        pltpu.sync_copy(nodes_hbm.at[pl.ds(s * nb, nb)], nodes2_v)
        pltpu.async_copy(adj_hbm.at[nodes2_v], neigh2_v, gsem).wait()

        def _flat(i, _):
            nf_v[1, pl.ds(i * L, L)] = neigh2_v[i]
            return 0
        lax.fori_loop(0, nb, _flat, 0)
        pltpu.sync_copy(ones_v.at[0, pl.ds(0, nb * DEG)],
                        present_sp.at[nf_v.at[1]], add=True)
        plsc.subcore_barrier()

        @pl.when(s == 0)
        def _enumerate():
            def _zu(i, _):
                uniq_v[pl.ds(i * L, L)] = jnp.zeros((L,), jnp.int32)
                return 0
            lax.fori_loop(0, U // L, _zu, 0)

            tmp_v[pl.ds(0, L)] = jnp.zeros((L,), jnp.int32)
            qlen = NPAD // 4
            base = jnp.zeros((L,), jnp.int32)
            for qc in range(4):
                pltpu.sync_copy(present_sp.at[pl.ds(qc * qlen, qlen)],
                                prescopy_v)

                def _enum(i, b, qc=qc):
                    pv = prescopy_v[pl.ds(i * L, L)]
                    m = pv > _f32(0.0)
                    mi = m.astype(jnp.int32)
                    inc = mi
                    for sh in (1, 2, 4, 8):
                        tmp_v[pl.ds(L, L)] = inc
                        shifted = plsc.load_gather(
                            tmp_v, [lane + (L - sh)])
                        inc = inc + shifted
                    tmp_v[pl.ds(L, L)] = inc
                    tot = plsc.load_gather(
                        tmp_v, [jnp.full((L,), 2 * L - 1, jnp.int32)])
                    rank = b + (inc - mi)
                    vals = lane + (qc * qlen + i * L)
                    plsc.store_scatter(uniq_v, [rank], vals, mask=m)
                    return b + tot
                base = lax.fori_loop(0, qlen // L, _enum, base)
            pltpu.sync_copy(uniq_v, uniq_sp)
            pltpu.sync_copy(uniq_v, uniq_hbm)
        plsc.subcore_barrier()

        for hf in range(2):
            pltpu.sync_copy(
                uniq_sp.at[pl.ds(s * (U // NS) + hf * (U // NS // 2),
                                 U // NS // 2)],
                myuniq_v)
            pltpu.async_copy(adj_hbm.at[myuniq_v], adj2_v, gsem).wait()
            for chunk in range(2):
                def _flat2(i, _):
                    af_v[chunk, pl.ds(i * L, L)] = adj2_v[chunk * 256 + i]
                    return 0
                lax.fori_loop(0, 256, _flat2, 0)
            for chunk in range(2):
                pltpu.sync_copy(ones_v.at[0],
                                colcount_sp.at[af_v.at[chunk]], add=True)
        plsc.subcore_barrier()

        @pl.when(s == 0)
        def _():
            pltpu.sync_copy(colcount_sp.at[pl.ds(0, N)], colcount_hbm)

    zdesc.wait()
    pltpu.sync_copy(val_v.at[0], mask_hbm.at[idx_v.at[0]])


def _sc1(nodes, adj, features_flat):
    mesh = plsc.VectorSubcoreMesh(core_axis_name="c", subcore_axis_name="s")
    kfn = pl.kernel(
        _sc1_body,
        compiler_params=_SC_PARAMS,
        out_type=[
            jax.ShapeDtypeStruct((B * N,), jnp.float32),
            jax.ShapeDtypeStruct((B, D), jnp.float32),
            jax.ShapeDtypeStruct((N,), jnp.float32),
            jax.ShapeDtypeStruct((U,), jnp.int32),
        ],
        mesh=mesh,
        scratch_types=[
            pltpu.VMEM((ROWS_PER_TILE,), jnp.int32),
            pltpu.VMEM((ROWS_PER_TILE, DEG), jnp.int32),
            pltpu.VMEM((NPAD,), jnp.float32),
            pltpu.VMEM((1, ROWS_PER_TILE * DEG), jnp.int32),
            pltpu.VMEM((1, ROWS_PER_TILE * DEG), jnp.float32),
            pltpu.VMEM((ROWS_PER_TILE, D), jnp.float32),
            pltpu.VMEM((ROWS_PER_TILE * DEG // 2, D), jnp.float32),
            pltpu.VMEM((B // NS,), jnp.int32),
            pltpu.VMEM((B // NS, DEG), jnp.int32),
            pltpu.VMEM((2, B // NS * DEG), jnp.int32),
            pltpu.VMEM((U // NS // 2,), jnp.int32),
            pltpu.VMEM((U // NS // 2, DEG), jnp.int32),
            pltpu.VMEM((2, 4096), jnp.int32),
            pltpu.VMEM((1, 4096), jnp.float32),
            pltpu.VMEM((NPAD // 4,), jnp.float32),
            pltpu.VMEM((U,), jnp.int32),
            pltpu.VMEM((2 * L,), jnp.int32),
            pltpu.VMEM((L,), jnp.int32),
            pltpu.VMEM_SHARED((ZELEMS,), jnp.float32),
            pltpu.VMEM_SHARED((NPAD,), jnp.float32),
            pltpu.VMEM_SHARED((NPAD,), jnp.float32),
            pltpu.VMEM_SHARED((U,), jnp.int32),
            pltpu.SemaphoreType.DMA,
            pltpu.SemaphoreType.DMA,
        ],
    )
    return kfn(nodes, adj, features_flat)


CH = 8
RING2 = 4
NCHUNK = N_PER_TILE // CH


def _sc2_body(scaled_hbm, adjf_hbm, table_hbm,
              adjs_v, ring_v, out_v, sems):
    c = lax.axis_index("c")
    s = lax.axis_index("s")
    wid = c * NS + s
    base_n = wid * N_PER_TILE

    pltpu.sync_copy(adjf_hbm.at[pl.ds(base_n * DEG, N_PER_TILE * DEG)],
                    adjs_v)

    rows = CH * DEG
    for r in range(RING2):
        pltpu.async_copy(scaled_hbm.at[adjs_v.at[pl.ds(r * rows, rows)]],
                         ring_v.at[r], sems.at[r])

    def _block(blk, _):
        for r in range(RING2):
            k = blk * RING2 + r
            pltpu.make_async_copy(
                scaled_hbm.at[adjs_v.at[pl.ds(r * rows, rows)]],
                ring_v.at[r], sems.at[r]).wait()
            for ul in range(CH):
                def _acc(j, acc):
                    return tuple(
                        acc[cc] + ring_v[r, ul * DEG + j, pl.ds(cc * L, L)]
                        for cc in range(D // L))
                acc0 = tuple(jnp.zeros((L,), jnp.float32)
                             for _ in range(D // L))
                acc = lax.fori_loop(0, DEG, _acc, acc0)
                u = k * CH + ul
                for cc in range(D // L):
                    out_v[u, pl.ds(cc * L, L)] = acc[cc]

            @pl.when(k + RING2 < NCHUNK)
            def _():
                pltpu.async_copy(
                    scaled_hbm.at[
                        adjs_v.at[pl.ds((k + RING2) * rows, rows)]],
                    ring_v.at[r], sems.at[r])
        return 0
    lax.fori_loop(0, NCHUNK // RING2, _block, 0)

    pltpu.sync_copy(out_v, table_hbm.at[pl.ds(base_n, N_PER_TILE)])


def _sc2(scaled, adj_flat_padded):
    mesh = plsc.VectorSubcoreMesh(core_axis_name="c", subcore_axis_name="s")
    kfn = pl.kernel(
        _sc2_body,
        compiler_params=_SC_PARAMS,
        out_type=jax.ShapeDtypeStruct((NPAD, D), jnp.float32),
        mesh=mesh,
        scratch_types=[
            pltpu.VMEM((N_PER_TILE * DEG,), jnp.int32),
            pltpu.VMEM((RING2, CH * DEG, D), jnp.float32),
            pltpu.VMEM((N_PER_TILE, D), jnp.float32),
            pltpu.SemaphoreType.DMA((RING2,)),
        ],
    )
    return kfn(scaled, adj_flat_padded)


def _sc3_body(emb_hbm, uniq_hbm, out_hbm, uniq_v, rows_v, sem):
    c = lax.axis_index("c")
    s = lax.axis_index("s")
    wid = c * NS + s
    base_u = wid * U_PER_TILE

    pltpu.sync_copy(uniq_hbm.at[pl.ds(base_u, U_PER_TILE)], uniq_v)
    pltpu.async_copy(emb_hbm.at[uniq_v], rows_v, sem).wait()
    pltpu.sync_copy(rows_v, out_hbm.at[pl.ds(base_u, U_PER_TILE)])


def _sc3(emb, uniq):
    mesh = plsc.VectorSubcoreMesh(core_axis_name="c", subcore_axis_name="s")
    kfn = pl.kernel(
        _sc3_body,
        compiler_params=_SC_PARAMS,
        out_type=jax.ShapeDtypeStruct((U, EMB), jnp.float32),
        mesh=mesh,
        scratch_types=[
            pltpu.VMEM((U_PER_TILE,), jnp.int32),
            pltpu.VMEM((U_PER_TILE, EMB), jnp.float32),
            pltpu.SemaphoreType.DMA,
        ],
    )
    return kfn(emb, uniq)


def _scale_body(feat_ref, cc_ref, out_ref):
    cc = jnp.maximum(cc_ref[...], 1.0)
    out_ref[...] = feat_ref[...] * lax.rsqrt(cc)


def _tc_scale(features, colcount2d):
    blk = 400
    return pl.pallas_call(
        _scale_body,
        grid=(N // blk,),
        in_specs=[
            pl.BlockSpec((blk, D), lambda i: (i, 0)),
            pl.BlockSpec((blk, 1), lambda i: (i, 0)),
        ],
        out_specs=pl.BlockSpec((blk, D), lambda i: (i, 0)),
        out_shape=jax.ShapeDtypeStruct((N, D), jnp.float32),
    )(features, colcount2d)


def _mm1_body(sum1_ref, w_ref, out_ref):
    a = sum1_ref[...] * (1.0 / DEG)
    out_ref[...] = jnp.maximum(
        jnp.dot(a, w_ref[...], preferred_element_type=jnp.float32), 0.0)


def _tc_mm1(sum1, weight):
    return pl.pallas_call(
        _mm1_body,
        out_shape=jax.ShapeDtypeStruct((B, EMB), jnp.float32),
    )(sum1, weight)


def _mm2_body(agg_ref, w_ref, out_ref):
    a = agg_ref[...] * 0.25
    out_ref[...] = jnp.maximum(
        jnp.dot(a, w_ref[...], preferred_element_type=jnp.float32), 0.0)


def _tc_mm2(table, weight):
    blk = 1024
    return pl.pallas_call(
        _mm2_body,
        grid=(NPAD // blk,),
        in_specs=[
            pl.BlockSpec((blk, D), lambda i: (i, 0)),
            pl.BlockSpec((D, EMB), lambda i: (0, 0)),
        ],
        out_specs=pl.BlockSpec((blk, EMB), lambda i: (i, 0)),
        out_shape=jax.ShapeDtypeStruct((NPAD, EMB), jnp.float32),
    )(table, weight)


def kernel(nodes, adj, features, weight):
    adj_flat = jnp.pad(adj, ((0, NPAD - N), (0, 0))).reshape(NPAD * DEG)
    mask_flat, sum1, colcount, uniq = _sc1(nodes, adj, features)
    scaled = _tc_scale(features, colcount.reshape(N, 1))
    to_feats = _tc_mm1(sum1, weight)
    table = _sc2(scaled, adj_flat)
    emb = _tc_mm2(table, weight)
    to_feats_neigh = _sc3(emb, uniq)
    return to_feats, to_feats_neigh, mask_flat.reshape(B, N)
